# baseline (speedup 1.0000x reference)
import functools

import jax
import jax.numpy as jnp
from jax import lax
from jax.experimental import pallas as pl
from jax.experimental.pallas import tpu as pltpu
from jax.experimental.pallas import tpu_sc as plsc

BATCH = 16384
HIST = 50
D = 16
N_FLAT = BATCH * HIST

NC = 2
NS = 16
NW = NC * NS
B_PER_W = N_FLAT // NW
CHUNK = 3200
N_CHUNKS = B_PER_W // CHUNK

_mesh = plsc.VectorSubcoreMesh(core_axis_name="c", subcore_axis_name="s")


@functools.partial(
    pl.kernel,
    mesh=_mesh,
    out_type=jax.ShapeDtypeStruct((N_FLAT, D), jnp.float32),
    scratch_types=[
        pltpu.VMEM((B_PER_W,), jnp.int32),
        pltpu.VMEM((CHUNK, D), jnp.float32),
        pltpu.VMEM((CHUNK, D), jnp.float32),
        pltpu.SemaphoreType.DMA,
        pltpu.SemaphoreType.DMA,
        pltpu.SemaphoreType.DMA,
        pltpu.SemaphoreType.DMA,
    ],
    compiler_params=pltpu.CompilerParams(use_tc_tiling_on_sc=False),
)
def _gather_kernel(idx_hbm, table_hbm, out_hbm, idx_v, rows0, rows1,
                   gs0, gs1, os0, os1):
    wid = lax.axis_index("s") * NC + lax.axis_index("c")
    base = pl.multiple_of(wid * B_PER_W, B_PER_W)
    pltpu.sync_copy(idx_hbm.at[pl.ds(base, B_PER_W)], idx_v)

    rows = (rows0, rows1)
    gsem = (gs0, gs1)
    osem = (os0, os1)

    def start_gather(g):
        return pltpu.async_copy(
            table_hbm.at[idx_v.at[pl.ds(g * CHUNK, CHUNK)]],
            rows[g % 2], gsem[g % 2])

    gathers = [None] * N_CHUNKS
    outs = [None] * N_CHUNKS
    gathers[0] = start_gather(0)
    for g in range(N_CHUNKS):
        if g + 1 < N_CHUNKS:
            if g >= 1:
                outs[g - 1].wait()
            gathers[g + 1] = start_gather(g + 1)
        gathers[g].wait()
        outs[g] = pltpu.async_copy(
            rows[g % 2], out_hbm.at[pl.ds(base + g * CHUNK, CHUNK)],
            osem[g % 2])
    outs[N_CHUNKS - 2].wait()
    outs[N_CHUNKS - 1].wait()


def _tc_transpose_body(i_ref, o_ref):
    x = i_ref[...]
    y = x.reshape(BATCH // 8, 8, D).transpose(2, 0, 1).reshape(D, BATCH)
    o_ref[...] = y


_tc_transpose = pl.pallas_call(
    _tc_transpose_body,
    grid=(HIST,),
    in_specs=[pl.BlockSpec((BATCH // 8, 8 * D), lambda i: (i, 0))],
    out_specs=pl.BlockSpec((D, BATCH), lambda i: (i, 0)),
    out_shape=jax.ShapeDtypeStruct((HIST * D, BATCH), jnp.float32),
)


def kernel(x, table):
    idx = x.T.reshape(N_FLAT).astype(jnp.int32)
    rows = _gather_kernel(idx, table)
    out2 = _tc_transpose(rows.reshape(HIST * BATCH // 8, 8 * D))
    return out2.reshape(HIST, D, BATCH).transpose(2, 0, 1)


# speedup vs baseline: 1.3674x; 1.3674x over previous
import functools

import jax
import jax.numpy as jnp
from jax import lax
from jax.experimental import pallas as pl
from jax.experimental.pallas import tpu as pltpu
from jax.experimental.pallas import tpu_sc as plsc

BATCH = 16384
HIST = 50
D = 16
N_FLAT = BATCH * HIST

NC = 2
NS = 16
NW = NC * NS
CPW = BATCH // 128 // NW
TOK = CPW * 128

_mesh = plsc.VectorSubcoreMesh(core_axis_name="c", subcore_axis_name="s")


@functools.partial(
    pl.kernel,
    mesh=_mesh,
    out_type=jax.ShapeDtypeStruct((HIST * 2 * 128, 1024), jnp.float32),
    scratch_types=[
        pltpu.VMEM((HIST * TOK,), jnp.int32),
        pltpu.VMEM((TOK, D), jnp.float32),
        pltpu.VMEM((TOK, D), jnp.float32),
        pltpu.SemaphoreType.DMA,
        pltpu.SemaphoreType.DMA,
        pltpu.SemaphoreType.DMA,
        pltpu.SemaphoreType.DMA,
        pltpu.SemaphoreType.DMA,
    ],
    compiler_params=pltpu.CompilerParams(use_tc_tiling_on_sc=False),
)
def _gather_kernel(idx_hbm, table_hbm, out_hbm, idx_v, rows0, rows1,
                   isem, gs0, gs1, os0, os1):
    wid = lax.axis_index("s") * NC + lax.axis_index("c")
    c0 = wid * CPW
    rows = (rows0, rows1)
    gsem = (gs0, gs1)
    osem = (os0, os1)

    idx_copies = []
    for h in range(HIST):
        off = pl.multiple_of(h * BATCH + c0 * 128, TOK)
        idx_copies.append(pltpu.async_copy(
            idx_hbm.at[pl.ds(off, TOK)],
            idx_v.at[pl.ds(h * TOK, TOK)], isem))
    for c in idx_copies:
        c.wait()

    def start_gather(h):
        return pltpu.async_copy(
            table_hbm.at[idx_v.at[pl.ds(h * TOK, TOK)]],
            rows[h % 2], gsem[h % 2])

    def start_outs(h):
        cps = []
        for g in range(2):
            base = pl.multiple_of(h * 256 + g * 128, 128)
            for cp in range(CPW):
                cps.append(pltpu.async_copy(
                    rows[h % 2].at[pl.ds(cp * 128, 128), pl.ds(g * 8, 8)],
                    out_hbm.at[pl.ds(base, 128), pl.ds((c0 + cp) * 8, 8)],
                    osem[h % 2]))
        return cps

    gathers = [None] * HIST
    outs = [None] * HIST
    gathers[0] = start_gather(0)
    for h in range(HIST):
        if h + 1 < HIST:
            if h >= 1:
                for c in outs[h - 1]:
                    c.wait()
            gathers[h + 1] = start_gather(h + 1)
        gathers[h].wait()
        outs[h] = start_outs(h)
    for c in outs[HIST - 2]:
        c.wait()
    for c in outs[HIST - 1]:
        c.wait()


def _tc_transpose_body(i_ref, o_ref):
    o_ref[...] = i_ref[...].T


_tc_transpose = pl.pallas_call(
    _tc_transpose_body,
    grid=(HIST * 2,),
    in_specs=[pl.BlockSpec((128, 1024), lambda i: (i, 0))],
    out_specs=pl.BlockSpec((1024, 128), lambda i: (i, 0)),
    out_shape=jax.ShapeDtypeStruct((HIST * 2 * 1024, 128), jnp.float32),
)


def kernel(x, table):
    idx = x.T.reshape(N_FLAT).astype(jnp.int32)
    slabs = _gather_kernel(idx, table)
    out2 = _tc_transpose(slabs)
    return (out2.reshape(HIST, 2, 128, 8, 128)
            .transpose(2, 4, 0, 1, 3).reshape(BATCH, HIST, D))
